# baseline (speedup 1.0000x reference)
import functools

import jax
import jax.numpy as jnp
from jax import lax
from jax.experimental import pallas as pl
from jax.experimental.pallas import tpu as pltpu
from jax.experimental.pallas import tpu_sc as plsc

_B, _D, _F, _E = 2048, 768, 2048, 8
_S = 2 * _B
_T = 256
_NT = _S // _T
_G = _NT + _E - 1
_LANES = 128
_SROWS = _S // _LANES
_DP = 1024


def _gate_body(x_ref, wg_ref, bg_ref, e_ref, w_ref):
    logits = jnp.dot(x_ref[...], wg_ref[...],
                     preferred_element_type=jnp.float32) + bg_ref[...]
    m = jnp.max(logits, axis=-1, keepdims=True)
    ex = jnp.exp(logits - m)
    w = ex / jnp.sum(ex, axis=-1, keepdims=True)
    lane = lax.broadcasted_iota(jnp.int32, w.shape, 1)
    m1 = jnp.max(w, axis=-1, keepdims=True)
    i1 = jnp.min(jnp.where(w >= m1, lane, jnp.int32(1 << 30)), axis=-1,
                 keepdims=True)
    wm = jnp.where(lane == i1, -1.0, w)
    m2 = jnp.max(wm, axis=-1, keepdims=True)
    i2 = jnp.min(jnp.where(wm >= m2, lane, jnp.int32(1 << 30)), axis=-1,
                 keepdims=True)
    lane8 = lax.broadcasted_iota(jnp.int32, e_ref.shape, 1)
    e_ref[...] = jnp.where(lane8 == 0, i1, 0) + jnp.where(lane8 == 1, i2, 0)
    w_ref[...] = (jnp.where(lane8 == 0, m1, 0.0)
                  + jnp.where(lane8 == 1, m2, 0.0))


def _sort_body(es_ref, pos_ref, meta_ref):
    es = es_ref[...]
    rr = lax.broadcasted_iota(jnp.int32, (_SROWS, _SROWS), 0)
    rc = lax.broadcasted_iota(jnp.int32, (_SROWS, _SROWS), 1)
    tl = jnp.where(rc < rr, 1.0, 0.0)
    ur = lax.broadcasted_iota(jnp.int32, (_LANES, _LANES), 0)
    uc = lax.broadcasted_iota(jnp.int32, (_LANES, _LANES), 1)
    uu = jnp.where(ur < uc, 1.0, 0.0)
    lane = lax.broadcasted_iota(jnp.int32, es.shape, 1)
    c_mat = jnp.zeros((_SROWS, _LANES), jnp.float32)
    for e in range(_E):
        mask = (es == e).astype(jnp.float32)
        c_mat = c_mat + jnp.sum(mask, axis=1, keepdims=True) * (
            jnp.where(lane == e, 1.0, 0.0))
    rowcum = jnp.dot(tl, c_mat, preferred_element_type=jnp.float32)
    tot = jnp.sum(c_mat, axis=0, keepdims=True)
    lane1 = lax.broadcasted_iota(jnp.int32, tot.shape, 1)
    offs = jnp.zeros(tot.shape, jnp.float32)
    run = jnp.zeros((1, 1), jnp.float32)
    for e in range(_E):
        offs = offs + jnp.where(lane1 == e, run, 0.0)
        run = run + jnp.sum(jnp.where(lane1 == e, tot, 0.0), axis=1,
                            keepdims=True)
    pos_f = jnp.zeros(es.shape, jnp.float32)
    for e in range(_E):
        mask = (es == e).astype(jnp.float32)
        pref = jnp.dot(mask, uu, preferred_element_type=jnp.float32)
        ext = jnp.sum(jnp.where(lane == e, rowcum, 0.0), axis=1,
                      keepdims=True)
        off_e = jnp.sum(jnp.where(lane1 == e, offs, 0.0), axis=1,
                        keepdims=True)
        pos_f = pos_f + mask * (pref + ext + off_e)
    pos_ref[...] = pos_f.astype(jnp.int32)

    def ext(v, e):
        return jnp.sum(jnp.where(lane1 == e, v, 0.0), axis=1, keepdims=True)

    tfl = float(_T)
    t0v = jnp.floor(offs / tfl)
    endv = offs + tot
    t1v = jnp.where(tot > 0, jnp.floor((endv - 1.0) / tfl), t0v - 1.0)
    niv = jnp.maximum(t1v - t0v + 1.0, 0.0)
    binc = jnp.zeros(tot.shape, jnp.float32)
    run2 = jnp.zeros((1, 1), jnp.float32)
    for e in range(_E):
        run2 = run2 + ext(niv, e)
        binc = binc + jnp.where(lane1 == e, run2, 0.0)
    bexc = binc - niv
    gf = lax.broadcasted_iota(jnp.int32, (1, _LANES), 1).astype(jnp.float32)
    e_of = jnp.zeros((1, _LANES), jnp.float32)
    for e in range(_E):
        e_of = e_of + jnp.where(gf >= ext(binc, e), 1.0, 0.0)
    e_of = jnp.minimum(e_of, float(_E - 1))
    valid = gf < run2
    t_of = jnp.zeros((1, _LANES), jnp.float32)
    lo_v = jnp.zeros((1, _LANES), jnp.float32)
    hi_v = jnp.zeros((1, _LANES), jnp.float32)
    for e in range(_E):
        sel = e_of == float(e)
        t_e = ext(t0v, e) + (gf - ext(bexc, e))
        t_of = t_of + jnp.where(sel, t_e, 0.0)
        lo_v = lo_v + jnp.where(
            sel, jnp.clip(ext(offs, e) - t_e * tfl, 0.0, tfl), 0.0)
        hi_v = hi_v + jnp.where(
            sel, jnp.clip(ext(endv, e) - t_e * tfl, 0.0, tfl), 0.0)
    t_of = jnp.where(valid, t_of, float(_NT - 1))
    e_sel = jnp.where(valid, e_of, float(_E - 1))
    lo_v = jnp.where(valid, lo_v, 0.0)
    hi_v = jnp.where(valid, hi_v, 0.0)

    row8 = lax.broadcasted_iota(jnp.int32, meta_ref.shape, 0)

    def put(r, v):
        return jnp.where(row8 == r, jnp.broadcast_to(v, meta_ref.shape), 0.0)

    meta_ref[...] = (put(0, offs) + put(1, tot) + put(2, t_of) + put(3, e_sel)
                     + put(4, lo_v) + put(5, hi_v)).astype(jnp.int32)


def _ffn_body(t_ref, e_ref, lo_ref, hi_ref, xs_ref, w1_ref, b1_ref, w2_ref,
              b2_ref, ys_ref):
    g = pl.program_id(0)
    h = jnp.dot(xs_ref[...], w1_ref[0], preferred_element_type=jnp.float32)
    h = jnp.maximum(h + b1_ref[0], 0.0)
    y = jnp.dot(h, w2_ref[0], preferred_element_type=jnp.float32) + b2_ref[0]
    row = lax.broadcasted_iota(jnp.int32, (_T, 1), 0)
    y = jnp.where((row >= lo_ref[g]) & (row < hi_ref[g]), y, 0.0)
    first = jnp.logical_or(g == 0, t_ref[g] != t_ref[jnp.maximum(g - 1, 0)])

    @pl.when(first)
    def _():
        ys_ref[...] = y

    @pl.when(jnp.logical_not(first))
    def _():
        ys_ref[...] += y


def _comb_body(a_ref, b_ref, wa_ref, wb_ref, o_ref):
    o_ref[...] = wa_ref[...] * a_ref[...] + wb_ref[...] * b_ref[...]


def _sc_dispatch(x, pos_slot):
    mesh = plsc.VectorSubcoreMesh(core_axis_name="c", subcore_axis_name="s")

    @functools.partial(
        pl.kernel, mesh=mesh,
        out_type=jax.ShapeDtypeStruct((_S, _D), jnp.float32),
        scratch_types=[pltpu.VMEM((64,), jnp.int32),
                       pltpu.VMEM((64,), jnp.int32),
                       pltpu.VMEM((64, _D), jnp.float32),
                       pltpu.VMEM((64, _D), jnp.float32),
                       pltpu.SemaphoreType.DMA,
                       pltpu.SemaphoreType.DMA,
                       pltpu.SemaphoreType.DMA,
                       pltpu.SemaphoreType.DMA],
    )
    def k(x_hbm, pos_hbm, xs_hbm, ia_v, ib_v, ra_v, rb_v, sa, sb, sc, sd):
        wid = lax.axis_index("s") * 2 + lax.axis_index("c")
        base = lax.rem(wid, 16) * _LANES
        pltpu.sync_copy(pos_hbm.at[wid, pl.ds(0, 64)], ia_v)
        pltpu.sync_copy(pos_hbm.at[wid, pl.ds(64, 64)], ib_v)
        ca = pltpu.async_copy(x_hbm.at[pl.ds(base, 64)], ra_v, sa)
        cb = pltpu.async_copy(x_hbm.at[pl.ds(base + 64, 64)], rb_v, sb)
        ca.wait()
        wa = pltpu.async_copy(ra_v, xs_hbm.at[ia_v], sc)
        cb.wait()
        wb = pltpu.async_copy(rb_v, xs_hbm.at[ib_v], sd)
        wa.wait()
        wb.wait()

    return k(x, pos_slot)


def _sc_combine(ys, pa, pb):
    mesh = plsc.VectorSubcoreMesh(core_axis_name="c", subcore_axis_name="s")
    otype = (jax.ShapeDtypeStruct((_B, _D), jnp.float32),
             jax.ShapeDtypeStruct((_B, _D), jnp.float32))

    @functools.partial(
        pl.kernel, mesh=mesh, out_type=otype,
        scratch_types=[pltpu.VMEM((64,), jnp.int32),
                       pltpu.VMEM((64,), jnp.int32),
                       pltpu.VMEM((64, _D), jnp.float32),
                       pltpu.VMEM((64, _D), jnp.float32),
                       pltpu.SemaphoreType.DMA,
                       pltpu.SemaphoreType.DMA,
                       pltpu.SemaphoreType.DMA,
                       pltpu.SemaphoreType.DMA],
    )
    def k(ys_hbm, pa_hbm, pb_hbm, a_hbm, b_hbm, ia_v, ib_v, ra_v, rb_v,
          sa, sb, sc, sd):
        wid = lax.axis_index("s") * 2 + lax.axis_index("c")
        pltpu.sync_copy(pa_hbm.at[wid], ia_v)
        pltpu.sync_copy(pb_hbm.at[wid], ib_v)
        ca = pltpu.async_copy(ys_hbm.at[ia_v], ra_v, sa)
        cb = pltpu.async_copy(ys_hbm.at[ib_v], rb_v, sb)
        ca.wait()
        wa = pltpu.async_copy(ra_v, a_hbm.at[pl.ds(wid * 64, 64)], sc)
        cb.wait()
        wb = pltpu.async_copy(rb_v, b_hbm.at[pl.ds(wid * 64, 64)], sd)
        wa.wait()
        wb.wait()

    return k(ys, pa, pb)


def kernel(x, Wg, bg, W1, b1, W2, b2):
    bt = _B // 8
    wg_pad = jnp.zeros((_D, _LANES), jnp.float32).at[:, :_E].set(Wg)
    bg_pad = jnp.full((1, _LANES), -1e30, jnp.float32).at[0, :_E].set(bg)

    eout, wout = pl.pallas_call(
        _gate_body,
        grid=(8,),
        in_specs=[
            pl.BlockSpec((bt, _D), lambda t: (t, 0)),
            pl.BlockSpec((_D, _LANES), lambda t: (0, 0)),
            pl.BlockSpec((1, _LANES), lambda t: (0, 0)),
        ],
        out_specs=[
            pl.BlockSpec((bt, _E), lambda t: (t, 0)),
            pl.BlockSpec((bt, _E), lambda t: (t, 0)),
        ],
        out_shape=(jax.ShapeDtypeStruct((_B, _E), jnp.int32),
                   jax.ShapeDtypeStruct((_B, _E), jnp.float32)),
    )(x, wg_pad, bg_pad)

    e_slot = jnp.concatenate([eout[:, 0], eout[:, 1]]).reshape(_SROWS, _LANES)

    pos_slot, meta = pl.pallas_call(
        _sort_body,
        out_shape=(jax.ShapeDtypeStruct((_SROWS, _LANES), jnp.int32),
                   jax.ShapeDtypeStruct((8, _LANES), jnp.int32)),
    )(e_slot)

    wi_t = meta[2, :_G]
    wi_e = meta[3, :_G]
    lo = meta[4, :_G]
    hi = meta[5, :_G]

    xs = _sc_dispatch(x, pos_slot)

    ys = pl.pallas_call(
        _ffn_body,
        grid_spec=pltpu.PrefetchScalarGridSpec(
            num_scalar_prefetch=4,
            grid=(_G,),
            in_specs=[
                pl.BlockSpec((_T, _D), lambda g, t, e, lo_, hi_: (t[g], 0)),
                pl.BlockSpec((1, _D, _F),
                             lambda g, t, e, lo_, hi_: (e[g], 0, 0)),
                pl.BlockSpec((1, 1, _F),
                             lambda g, t, e, lo_, hi_: (e[g], 0, 0)),
                pl.BlockSpec((1, _F, _D),
                             lambda g, t, e, lo_, hi_: (e[g], 0, 0)),
                pl.BlockSpec((1, 1, _D),
                             lambda g, t, e, lo_, hi_: (e[g], 0, 0)),
            ],
            out_specs=pl.BlockSpec((_T, _D),
                                   lambda g, t, e, lo_, hi_: (t[g], 0)),
        ),
        out_shape=jax.ShapeDtypeStruct((_S, _D), jnp.float32),
        compiler_params=pltpu.CompilerParams(
            dimension_semantics=("arbitrary",)),
    )(wi_t, wi_e, lo, hi,
      xs, W1, b1.reshape(_E, 1, _F), W2, b2.reshape(_E, 1, _D))

    pa = pos_slot[:_SROWS // 2].reshape(32, 64)
    pb = pos_slot[_SROWS // 2:].reshape(32, 64)
    a_rows, b_rows = _sc_combine(ys, pa, pb)

    out = pl.pallas_call(
        _comb_body,
        grid=(4,),
        in_specs=[
            pl.BlockSpec((_B // 4, _D), lambda t: (t, 0)),
            pl.BlockSpec((_B // 4, _D), lambda t: (t, 0)),
            pl.BlockSpec((_B // 4, 1), lambda t: (t, 0)),
            pl.BlockSpec((_B // 4, 1), lambda t: (t, 0)),
        ],
        out_specs=pl.BlockSpec((_B // 4, _D), lambda t: (t, 0)),
        out_shape=jax.ShapeDtypeStruct((_B, _D), jnp.float32),
    )(a_rows, b_rows, wout[:, 0:1], wout[:, 1:2])
    return out


# speedup vs baseline: 1.0283x; 1.0283x over previous
import functools

import jax
import jax.numpy as jnp
from jax import lax
from jax.experimental import pallas as pl
from jax.experimental.pallas import tpu as pltpu
from jax.experimental.pallas import tpu_sc as plsc

_B, _D, _F, _E = 2048, 768, 2048, 8
_S = 2 * _B
_T = 512
_NT = _S // _T
_G = _NT + _E - 1
_LANES = 128
_SROWS = _S // _LANES


def _gate_body(x_ref, wg_ref, bg_ref, e_ref, w_ref):
    logits = jnp.dot(x_ref[...], wg_ref[...],
                     preferred_element_type=jnp.float32) + bg_ref[...]
    m = jnp.max(logits, axis=-1, keepdims=True)
    ex = jnp.exp(logits - m)
    w = ex / jnp.sum(ex, axis=-1, keepdims=True)
    lane = lax.broadcasted_iota(jnp.int32, w.shape, 1)
    m1 = jnp.max(w, axis=-1, keepdims=True)
    i1 = jnp.min(jnp.where(w >= m1, lane, jnp.int32(1 << 30)), axis=-1,
                 keepdims=True)
    wm = jnp.where(lane == i1, -1.0, w)
    m2 = jnp.max(wm, axis=-1, keepdims=True)
    i2 = jnp.min(jnp.where(wm >= m2, lane, jnp.int32(1 << 30)), axis=-1,
                 keepdims=True)
    lane8 = lax.broadcasted_iota(jnp.int32, e_ref.shape, 1)
    e_ref[...] = jnp.where(lane8 == 0, i1, 0) + jnp.where(lane8 == 1, i2, 0)
    w_ref[...] = (jnp.where(lane8 == 0, m1, 0.0)
                  + jnp.where(lane8 == 1, m2, 0.0))


def _sort_body(es_ref, pos_ref, meta_ref):
    es = es_ref[...]
    rr = lax.broadcasted_iota(jnp.int32, (_SROWS, _SROWS), 0)
    rc = lax.broadcasted_iota(jnp.int32, (_SROWS, _SROWS), 1)
    tl = jnp.where(rc < rr, 1.0, 0.0)
    ur = lax.broadcasted_iota(jnp.int32, (_LANES, _LANES), 0)
    uc = lax.broadcasted_iota(jnp.int32, (_LANES, _LANES), 1)
    uu = jnp.where(ur < uc, 1.0, 0.0)
    lane = lax.broadcasted_iota(jnp.int32, es.shape, 1)
    c_mat = jnp.zeros((_SROWS, _LANES), jnp.float32)
    for e in range(_E):
        mask = (es == e).astype(jnp.float32)
        c_mat = c_mat + jnp.sum(mask, axis=1, keepdims=True) * (
            jnp.where(lane == e, 1.0, 0.0))
    rowcum = jnp.dot(tl, c_mat, preferred_element_type=jnp.float32)
    tot = jnp.sum(c_mat, axis=0, keepdims=True)
    lane1 = lax.broadcasted_iota(jnp.int32, tot.shape, 1)
    offs = jnp.zeros(tot.shape, jnp.float32)
    run = jnp.zeros((1, 1), jnp.float32)
    for e in range(_E):
        offs = offs + jnp.where(lane1 == e, run, 0.0)
        run = run + jnp.sum(jnp.where(lane1 == e, tot, 0.0), axis=1,
                            keepdims=True)
    pos_f = jnp.zeros(es.shape, jnp.float32)
    for e in range(_E):
        mask = (es == e).astype(jnp.float32)
        pref = jnp.dot(mask, uu, preferred_element_type=jnp.float32)
        ext = jnp.sum(jnp.where(lane == e, rowcum, 0.0), axis=1,
                      keepdims=True)
        off_e = jnp.sum(jnp.where(lane1 == e, offs, 0.0), axis=1,
                        keepdims=True)
        pos_f = pos_f + mask * (pref + ext + off_e)
    pos_ref[...] = pos_f.astype(jnp.int32)

    def ext(v, e):
        return jnp.sum(jnp.where(lane1 == e, v, 0.0), axis=1, keepdims=True)

    tfl = float(_T)
    t0v = jnp.floor(offs / tfl)
    endv = offs + tot
    t1v = jnp.where(tot > 0, jnp.floor((endv - 1.0) / tfl), t0v - 1.0)
    niv = jnp.maximum(t1v - t0v + 1.0, 0.0)
    binc = jnp.zeros(tot.shape, jnp.float32)
    run2 = jnp.zeros((1, 1), jnp.float32)
    for e in range(_E):
        run2 = run2 + ext(niv, e)
        binc = binc + jnp.where(lane1 == e, run2, 0.0)
    bexc = binc - niv
    gf = lax.broadcasted_iota(jnp.int32, (1, _LANES), 1).astype(jnp.float32)
    e_of = jnp.zeros((1, _LANES), jnp.float32)
    for e in range(_E):
        e_of = e_of + jnp.where(gf >= ext(binc, e), 1.0, 0.0)
    e_of = jnp.minimum(e_of, float(_E - 1))
    valid = gf < run2
    t_of = jnp.zeros((1, _LANES), jnp.float32)
    lo_v = jnp.zeros((1, _LANES), jnp.float32)
    hi_v = jnp.zeros((1, _LANES), jnp.float32)
    for e in range(_E):
        sel = e_of == float(e)
        t_e = ext(t0v, e) + (gf - ext(bexc, e))
        t_of = t_of + jnp.where(sel, t_e, 0.0)
        lo_v = lo_v + jnp.where(
            sel, jnp.clip(ext(offs, e) - t_e * tfl, 0.0, tfl), 0.0)
        hi_v = hi_v + jnp.where(
            sel, jnp.clip(ext(endv, e) - t_e * tfl, 0.0, tfl), 0.0)
    t_of = jnp.where(valid, t_of, float(_NT - 1))
    e_sel = jnp.where(valid, e_of, float(_E - 1))
    lo_v = jnp.where(valid, lo_v, 0.0)
    hi_v = jnp.where(valid, hi_v, 0.0)

    row8 = lax.broadcasted_iota(jnp.int32, meta_ref.shape, 0)

    def put(r, v):
        return jnp.where(row8 == r, jnp.broadcast_to(v, meta_ref.shape), 0.0)

    meta_ref[...] = (put(0, offs) + put(1, tot) + put(2, t_of) + put(3, e_sel)
                     + put(4, lo_v) + put(5, hi_v)).astype(jnp.int32)


def _ffn_body(t_ref, e_ref, lo_ref, hi_ref, xs_ref, w1_ref, b1_ref, w2_ref,
              b2_ref, ys_ref):
    g = pl.program_id(0)
    h = jnp.dot(xs_ref[...], w1_ref[0], preferred_element_type=jnp.float32)
    h = jnp.maximum(h + b1_ref[0], 0.0)
    y = jnp.dot(h, w2_ref[0], preferred_element_type=jnp.float32) + b2_ref[0]
    row = lax.broadcasted_iota(jnp.int32, (_T, 1), 0)
    y = jnp.where((row >= lo_ref[g]) & (row < hi_ref[g]), y, 0.0)
    first = jnp.logical_or(g == 0, t_ref[g] != t_ref[jnp.maximum(g - 1, 0)])

    @pl.when(first)
    def _():
        ys_ref[...] = y

    @pl.when(jnp.logical_not(first))
    def _():
        ys_ref[...] += y


def _comb_body(a_ref, b_ref, wa_ref, wb_ref, o_ref):
    o_ref[...] = wa_ref[...] * a_ref[...] + wb_ref[...] * b_ref[...]


def _sc_dispatch(x, pos_slot):
    mesh = plsc.VectorSubcoreMesh(core_axis_name="c", subcore_axis_name="s")

    @functools.partial(
        pl.kernel, mesh=mesh,
        out_type=jax.ShapeDtypeStruct((_S, _D), jnp.float32),
        scratch_types=[pltpu.VMEM((64,), jnp.int32),
                       pltpu.VMEM((64,), jnp.int32),
                       pltpu.VMEM((64, _D), jnp.float32),
                       pltpu.VMEM((64, _D), jnp.float32),
                       pltpu.SemaphoreType.DMA,
                       pltpu.SemaphoreType.DMA,
                       pltpu.SemaphoreType.DMA,
                       pltpu.SemaphoreType.DMA],
    )
    def k(x_hbm, pos_hbm, xs_hbm, ia_v, ib_v, ra_v, rb_v, sa, sb, sc, sd):
        wid = lax.axis_index("s") * 2 + lax.axis_index("c")
        base = lax.rem(wid, 16) * _LANES
        pltpu.sync_copy(pos_hbm.at[wid, pl.ds(0, 64)], ia_v)
        pltpu.sync_copy(pos_hbm.at[wid, pl.ds(64, 64)], ib_v)
        ca = pltpu.async_copy(x_hbm.at[pl.ds(base, 64)], ra_v, sa)
        cb = pltpu.async_copy(x_hbm.at[pl.ds(base + 64, 64)], rb_v, sb)
        ca.wait()
        wa = pltpu.async_copy(ra_v, xs_hbm.at[ia_v], sc)
        cb.wait()
        wb = pltpu.async_copy(rb_v, xs_hbm.at[ib_v], sd)
        wa.wait()
        wb.wait()

    return k(x, pos_slot)


def _sc_combine(ys, pa, pb):
    mesh = plsc.VectorSubcoreMesh(core_axis_name="c", subcore_axis_name="s")
    otype = (jax.ShapeDtypeStruct((_B, _D), jnp.float32),
             jax.ShapeDtypeStruct((_B, _D), jnp.float32))

    @functools.partial(
        pl.kernel, mesh=mesh, out_type=otype,
        scratch_types=[pltpu.VMEM((64,), jnp.int32),
                       pltpu.VMEM((64,), jnp.int32),
                       pltpu.VMEM((64, _D), jnp.float32),
                       pltpu.VMEM((64, _D), jnp.float32),
                       pltpu.SemaphoreType.DMA,
                       pltpu.SemaphoreType.DMA,
                       pltpu.SemaphoreType.DMA,
                       pltpu.SemaphoreType.DMA],
    )
    def k(ys_hbm, pa_hbm, pb_hbm, a_hbm, b_hbm, ia_v, ib_v, ra_v, rb_v,
          sa, sb, sc, sd):
        wid = lax.axis_index("s") * 2 + lax.axis_index("c")
        pltpu.sync_copy(pa_hbm.at[wid], ia_v)
        pltpu.sync_copy(pb_hbm.at[wid], ib_v)
        ca = pltpu.async_copy(ys_hbm.at[ia_v], ra_v, sa)
        cb = pltpu.async_copy(ys_hbm.at[ib_v], rb_v, sb)
        ca.wait()
        wa = pltpu.async_copy(ra_v, a_hbm.at[pl.ds(wid * 64, 64)], sc)
        cb.wait()
        wb = pltpu.async_copy(rb_v, b_hbm.at[pl.ds(wid * 64, 64)], sd)
        wa.wait()
        wb.wait()

    return k(ys, pa, pb)


def kernel(x, Wg, bg, W1, b1, W2, b2):
    bt = _B // 8
    wg_pad = jnp.zeros((_D, _LANES), jnp.float32).at[:, :_E].set(Wg)
    bg_pad = jnp.full((1, _LANES), -1e30, jnp.float32).at[0, :_E].set(bg)

    eout, wout = pl.pallas_call(
        _gate_body,
        grid=(8,),
        in_specs=[
            pl.BlockSpec((bt, _D), lambda t: (t, 0)),
            pl.BlockSpec((_D, _LANES), lambda t: (0, 0)),
            pl.BlockSpec((1, _LANES), lambda t: (0, 0)),
        ],
        out_specs=[
            pl.BlockSpec((bt, _E), lambda t: (t, 0)),
            pl.BlockSpec((bt, _E), lambda t: (t, 0)),
        ],
        out_shape=(jax.ShapeDtypeStruct((_B, _E), jnp.int32),
                   jax.ShapeDtypeStruct((_B, _E), jnp.float32)),
    )(x, wg_pad, bg_pad)

    e_slot = jnp.concatenate([eout[:, 0], eout[:, 1]]).reshape(_SROWS, _LANES)

    pos_slot, meta = pl.pallas_call(
        _sort_body,
        out_shape=(jax.ShapeDtypeStruct((_SROWS, _LANES), jnp.int32),
                   jax.ShapeDtypeStruct((8, _LANES), jnp.int32)),
    )(e_slot)

    wi_t = meta[2, :_G]
    wi_e = meta[3, :_G]
    lo = meta[4, :_G]
    hi = meta[5, :_G]

    xs = _sc_dispatch(x, pos_slot)

    ys = pl.pallas_call(
        _ffn_body,
        grid_spec=pltpu.PrefetchScalarGridSpec(
            num_scalar_prefetch=4,
            grid=(_G,),
            in_specs=[
                pl.BlockSpec((_T, _D), lambda g, t, e, lo_, hi_: (t[g], 0)),
                pl.BlockSpec((1, _D, _F),
                             lambda g, t, e, lo_, hi_: (e[g], 0, 0)),
                pl.BlockSpec((1, 1, _F),
                             lambda g, t, e, lo_, hi_: (e[g], 0, 0)),
                pl.BlockSpec((1, _F, _D),
                             lambda g, t, e, lo_, hi_: (e[g], 0, 0)),
                pl.BlockSpec((1, 1, _D),
                             lambda g, t, e, lo_, hi_: (e[g], 0, 0)),
            ],
            out_specs=pl.BlockSpec((_T, _D),
                                   lambda g, t, e, lo_, hi_: (t[g], 0)),
        ),
        out_shape=jax.ShapeDtypeStruct((_S, _D), jnp.float32),
        compiler_params=pltpu.CompilerParams(
            dimension_semantics=("arbitrary",)),
    )(wi_t, wi_e, lo, hi,
      xs, W1, b1.reshape(_E, 1, _F), W2, b2.reshape(_E, 1, _D))

    pa = pos_slot[:_SROWS // 2].reshape(32, 64)
    pb = pos_slot[_SROWS // 2:].reshape(32, 64)
    a_rows, b_rows = _sc_combine(ys, pa, pb)

    out = pl.pallas_call(
        _comb_body,
        grid=(4,),
        in_specs=[
            pl.BlockSpec((_B // 4, _D), lambda t: (t, 0)),
            pl.BlockSpec((_B // 4, _D), lambda t: (t, 0)),
            pl.BlockSpec((_B // 4, 1), lambda t: (t, 0)),
            pl.BlockSpec((_B // 4, 1), lambda t: (t, 0)),
        ],
        out_specs=pl.BlockSpec((_B // 4, _D), lambda t: (t, 0)),
        out_shape=jax.ShapeDtypeStruct((_B, _D), jnp.float32),
    )(a_rows, b_rows, wout[:, 0:1], wout[:, 1:2])
    return out
